# baseline (speedup 1.0000x reference)
import functools

import jax
import jax.numpy as jnp
from jax import lax
from jax.experimental import pallas as pl
from jax.experimental.pallas import tpu as pltpu
from jax.experimental.pallas import tpu_sc as plsc

N_F = 40960
D = 512
BATCH = 16384
NC = 2
NS = 16
NW = NC * NS
PW = BATCH // NW
GRP = 16
GROUPS = PW // GRP
NCH = D // 32


def _clipsq(x):
    y = jnp.minimum(jnp.maximum(x, 0.0), 1.0)
    return y * y


def _split_eo(v):
    return plsc.unpack(v, format=plsc.PackFormat.INTERLEAVED)


def _make_sc_kernel():
    mesh = plsc.VectorSubcoreMesh(core_axis_name="c", subcore_axis_name="s")

    @functools.partial(
        pl.kernel,
        mesh=mesh,
        out_type=jax.ShapeDtypeStruct((BATCH,), jnp.float32),
        compiler_params=pltpu.CompilerParams(
            needs_layout_passes=False, use_tc_tiling_on_sc=False),
        scratch_types=[
            pltpu.VMEM((PW, 64), jnp.int32),
            pltpu.VMEM((PW,), jnp.float32),
            pltpu.VMEM((PW,), jnp.float32),
            pltpu.VMEM((N_F + 1,), jnp.float32),
            pltpu.VMEM((64, D), jnp.bfloat16),
            pltpu.VMEM((64, D), jnp.bfloat16),
            pltpu.VMEM((D,), jnp.float32),
            pltpu.VMEM((D,), jnp.float32),
            pltpu.VMEM((D,), jnp.float32),
            pltpu.VMEM((NCH, 16), jnp.float32),
            pltpu.VMEM((NCH, 16), jnp.float32),
            pltpu.VMEM((NCH, 16), jnp.float32),
            pltpu.VMEM((NCH, 16), jnp.float32),
            pltpu.VMEM((NCH, 16), jnp.float32),
            pltpu.VMEM((NCH, 16), jnp.float32),
            pltpu.VMEM((16,), jnp.float32),
            pltpu.SemaphoreType.DMA,
            pltpu.SemaphoreType.DMA,
        ],
    )
    def sc_kernel(table_hbm, wft_hbm, bft_hbm, stm_hbm, psqt_hbm,
                  bias_hbm, w1_hbm, w2_hbm, fcb_hbm, out_hbm,
                  idx_v, stm_v, out_v, psqt_v, rows0, rows1,
                  bias_v, w1_v, w2_v,
                  be_v, bo_v, w1e_v, w1o_v, w2e_v, w2o_v, fcb_v, sem0, sem1):
        wid = lax.axis_index("s") * NC + lax.axis_index("c")
        base = wid * PW
        pltpu.sync_copy(wft_hbm.at[pl.ds(base, PW)], idx_v.at[:, pl.ds(0, 32)])
        pltpu.sync_copy(bft_hbm.at[pl.ds(base, PW)], idx_v.at[:, pl.ds(32, 32)])

        def plus1_body(r, carry):
            for c in range(4):
                sl = pl.ds(c * 16, 16)
                idx_v[r, sl] = idx_v[r, sl] + 1
            return carry

        lax.fori_loop(0, PW, plus1_body, 0)
        pltpu.sync_copy(stm_hbm.at[pl.ds(base, PW)], stm_v)
        pltpu.sync_copy(psqt_hbm, psqt_v)
        pltpu.sync_copy(bias_hbm, bias_v)
        pltpu.sync_copy(w1_hbm, w1_v)
        pltpu.sync_copy(w2_hbm, w2_v)
        pltpu.sync_copy(fcb_hbm, fcb_v)

        rows = (rows0, rows1)
        sems = (sem0, sem1)
        lanes = lax.iota(jnp.int32, 16)
        zero16 = jnp.zeros((16,), jnp.float32)

        for j in range(NCH):
            ii = lanes * 2 + (32 * j)
            be_v[j, :] = plsc.load_gather(bias_v, [ii])
            bo_v[j, :] = plsc.load_gather(bias_v, [ii + 1])
            w1e_v[j, :] = plsc.load_gather(w1_v, [ii])
            w1o_v[j, :] = plsc.load_gather(w1_v, [ii + 1])
            w2e_v[j, :] = plsc.load_gather(w2_v, [ii])
            w2o_v[j, :] = plsc.load_gather(w2_v, [ii + 1])

        def issue(i, b):
            return pltpu.async_copy(table_hbm.at[idx_v.at[i]], rows[b], sems[b])

        def wait(i, b):
            pltpu.make_async_copy(
                table_hbm.at[idx_v.at[i]], rows[b], sems[b]).wait()

        issue(0, 0)
        issue(1, 1)

        def group_body(g, carry):
            s0 = g * GRP
            av = zero16
            bv = zero16
            pv = zero16
            for k in range(GRP):
                b = k % 2
                i = s0 + k
                wait(i, b)
                r_ref = rows[b]

                def jbody(j, acc, r_ref=r_ref):
                    cw1, cw2, cb1, cb2 = acc
                    cds = pl.ds(j * 32, 32)

                    def side(brow):
                        parts = []
                        for p in range(4):
                            a = r_ref[brow + p, cds]
                            for r in range(p + 4, 32, 4):
                                a = a + r_ref[brow + r, cds]
                            parts.append(a)
                        s = (parts[0] + parts[1]) + (parts[2] + parts[3])
                        return _split_eo(s)

                    swe, swo = side(0)
                    sbe, sbo = side(32)
                    bej = be_v[j]
                    boj = bo_v[j]
                    xwe = _clipsq(swe + bej)
                    xwo = _clipsq(swo + boj)
                    xbe = _clipsq(sbe + bej)
                    xbo = _clipsq(sbo + boj)
                    w1ej = w1e_v[j]
                    w1oj = w1o_v[j]
                    w2ej = w2e_v[j]
                    w2oj = w2o_v[j]
                    cw1 = cw1 + xwe * w1ej + xwo * w1oj
                    cw2 = cw2 + xwe * w2ej + xwo * w2oj
                    cb1 = cb1 + xbe * w1ej + xbo * w1oj
                    cb2 = cb2 + xbe * w2ej + xbo * w2oj
                    return (cw1, cw2, cb1, cb2)

                cw1, cw2, cb1, cb2 = lax.fori_loop(
                    0, NCH, jbody, (zero16, zero16, zero16, zero16))
                a_s = jnp.sum(cw1 + cb2)
                b_s = jnp.sum(cb1 + cw2)
                g0 = plsc.load_gather(psqt_v, [idx_v[i, pl.ds(0, 16)]])
                g1 = plsc.load_gather(psqt_v, [idx_v[i, pl.ds(16, 16)]])
                g2 = plsc.load_gather(psqt_v, [idx_v[i, pl.ds(32, 16)]])
                g3 = plsc.load_gather(psqt_v, [idx_v[i, pl.ds(48, 16)]])
                p_s = jnp.sum((g0 + g1) - (g2 + g3))
                m = lanes == k
                av = jnp.where(m, a_s, av)
                bv = jnp.where(m, b_s, bv)
                pv = jnp.where(m, p_s, pv)
                issue(jnp.minimum(i + 2, PW - 1), b)
            sv = stm_v[pl.ds(s0, GRP)]
            ov = av + sv * (bv - av) + pv * (0.5 - sv) + fcb_v[...]
            out_v[pl.ds(s0, GRP)] = ov
            return carry

        lax.fori_loop(0, GROUPS, group_body, 0)
        wait(PW - 1, 0)
        wait(PW - 1, 1)
        pltpu.sync_copy(out_v, out_hbm.at[pl.ds(base, PW)])

    return sc_kernel


_sc_kernel = _make_sc_kernel()


def kernel(wft_ics, bft_ics, stm, ft_weight, ft_bias, psqt_weight, fc_w, fc_b):
    table_bf = ft_weight.astype(jnp.bfloat16)
    psqt_f = psqt_weight.reshape(-1)
    stm_f = stm.reshape(-1)
    w1 = fc_w[0, :D]
    w2 = fc_w[0, D:]
    fcb = jnp.broadcast_to(fc_b, (16,)).astype(jnp.float32)
    out = _sc_kernel(table_bf, wft_ics, bft_ics, stm_f, psqt_f,
                     ft_bias, w1, w2, fcb)
    return out.reshape(BATCH, 1)


# speedup vs baseline: 1.0026x; 1.0026x over previous
import functools

import jax
import jax.numpy as jnp
from jax import lax
from jax.experimental import pallas as pl
from jax.experimental.pallas import tpu as pltpu
from jax.experimental.pallas import tpu_sc as plsc

N_F = 40960
D = 512
BATCH = 16384
NC = 2
NS = 16
NW = NC * NS
PW = BATCH // NW
GRP = 16
GROUPS = PW // GRP
NCH = D // 32


def _clipsq(x):
    y = jnp.minimum(jnp.maximum(x, 0.0), 1.0)
    return y * y


def _split_eo(v):
    return plsc.unpack(v, format=plsc.PackFormat.INTERLEAVED)


def _make_sc_kernel():
    mesh = plsc.VectorSubcoreMesh(core_axis_name="c", subcore_axis_name="s")

    @functools.partial(
        pl.kernel,
        mesh=mesh,
        out_type=jax.ShapeDtypeStruct((BATCH,), jnp.float32),
        compiler_params=pltpu.CompilerParams(
            needs_layout_passes=False, use_tc_tiling_on_sc=False),
        scratch_types=[
            pltpu.VMEM((PW, 64), jnp.int32),
            pltpu.VMEM((PW,), jnp.float32),
            pltpu.VMEM((PW,), jnp.float32),
            pltpu.VMEM((N_F + 1,), jnp.float32),
            pltpu.VMEM((64, D), jnp.bfloat16),
            pltpu.VMEM((64, D), jnp.bfloat16),
            pltpu.VMEM((D,), jnp.float32),
            pltpu.VMEM((D,), jnp.float32),
            pltpu.VMEM((D,), jnp.float32),
            pltpu.VMEM((NCH, 16), jnp.float32),
            pltpu.VMEM((NCH, 16), jnp.float32),
            pltpu.VMEM((NCH, 16), jnp.float32),
            pltpu.VMEM((NCH, 16), jnp.float32),
            pltpu.VMEM((NCH, 16), jnp.float32),
            pltpu.VMEM((NCH, 16), jnp.float32),
            pltpu.VMEM((16,), jnp.float32),
            pltpu.SemaphoreType.DMA,
            pltpu.SemaphoreType.DMA,
        ],
    )
    def sc_kernel(table_hbm, wft_hbm, bft_hbm, stm_hbm, psqt_hbm,
                  bias_hbm, w1_hbm, w2_hbm, fcb_hbm, out_hbm,
                  idx_v, stm_v, out_v, psqt_v, rows0, rows1,
                  bias_v, w1_v, w2_v,
                  be_v, bo_v, w1e_v, w1o_v, w2e_v, w2o_v, fcb_v, sem0, sem1):
        wid = lax.axis_index("s") * NC + lax.axis_index("c")
        base = wid * PW
        pltpu.sync_copy(wft_hbm.at[pl.ds(base, PW)], idx_v.at[:, pl.ds(0, 32)])
        pltpu.sync_copy(bft_hbm.at[pl.ds(base, PW)], idx_v.at[:, pl.ds(32, 32)])
        pltpu.sync_copy(stm_hbm.at[pl.ds(base, PW)], stm_v)
        pltpu.sync_copy(psqt_hbm, psqt_v)
        pltpu.sync_copy(bias_hbm, bias_v)
        pltpu.sync_copy(w1_hbm, w1_v)
        pltpu.sync_copy(w2_hbm, w2_v)
        pltpu.sync_copy(fcb_hbm, fcb_v)

        rows = (rows0, rows1)
        sems = (sem0, sem1)
        lanes = lax.iota(jnp.int32, 16)
        zero16 = jnp.zeros((16,), jnp.float32)

        for j in range(NCH):
            ii = lanes * 2 + (32 * j)
            be_v[j, :] = plsc.load_gather(bias_v, [ii])
            bo_v[j, :] = plsc.load_gather(bias_v, [ii + 1])
            w1e_v[j, :] = plsc.load_gather(w1_v, [ii])
            w1o_v[j, :] = plsc.load_gather(w1_v, [ii + 1])
            w2e_v[j, :] = plsc.load_gather(w2_v, [ii])
            w2o_v[j, :] = plsc.load_gather(w2_v, [ii + 1])

        tbl = table_hbm.at[pl.ds(1, N_F)]

        def issue(i, b):
            return pltpu.async_copy(tbl.at[idx_v.at[i]], rows[b], sems[b])

        def wait(i, b):
            pltpu.make_async_copy(
                tbl.at[idx_v.at[i]], rows[b], sems[b]).wait()

        issue(0, 0)
        issue(1, 1)

        def group_body(g, carry):
            s0 = g * GRP
            av = zero16
            bv = zero16
            pv = zero16
            for k in range(GRP):
                b = k % 2
                i = s0 + k
                wait(i, b)
                r_ref = rows[b]

                def jbody(j, acc, r_ref=r_ref):
                    cw1, cw2, cb1, cb2 = acc
                    cds = pl.ds(j * 32, 32)

                    def side(brow):
                        parts = []
                        for p in range(4):
                            a = r_ref[brow + p, cds]
                            for r in range(p + 4, 32, 4):
                                a = a + r_ref[brow + r, cds]
                            parts.append(a)
                        s = (parts[0] + parts[1]) + (parts[2] + parts[3])
                        return _split_eo(s)

                    swe, swo = side(0)
                    sbe, sbo = side(32)
                    bej = be_v[j]
                    boj = bo_v[j]
                    xwe = _clipsq(swe + bej)
                    xwo = _clipsq(swo + boj)
                    xbe = _clipsq(sbe + bej)
                    xbo = _clipsq(sbo + boj)
                    w1ej = w1e_v[j]
                    w1oj = w1o_v[j]
                    w2ej = w2e_v[j]
                    w2oj = w2o_v[j]
                    cw1 = cw1 + xwe * w1ej + xwo * w1oj
                    cw2 = cw2 + xwe * w2ej + xwo * w2oj
                    cb1 = cb1 + xbe * w1ej + xbo * w1oj
                    cb2 = cb2 + xbe * w2ej + xbo * w2oj
                    return (cw1, cw2, cb1, cb2)

                cw1, cw2, cb1, cb2 = lax.fori_loop(
                    0, NCH, jbody, (zero16, zero16, zero16, zero16))
                a_s = jnp.sum(cw1 + cb2)
                b_s = jnp.sum(cb1 + cw2)
                g0 = plsc.load_gather(psqt_v, [idx_v[i, pl.ds(0, 16)] + 1])
                g1 = plsc.load_gather(psqt_v, [idx_v[i, pl.ds(16, 16)] + 1])
                g2 = plsc.load_gather(psqt_v, [idx_v[i, pl.ds(32, 16)] + 1])
                g3 = plsc.load_gather(psqt_v, [idx_v[i, pl.ds(48, 16)] + 1])
                p_s = jnp.sum((g0 + g1) - (g2 + g3))
                m = lanes == k
                av = jnp.where(m, a_s, av)
                bv = jnp.where(m, b_s, bv)
                pv = jnp.where(m, p_s, pv)
                issue(jnp.minimum(i + 2, PW - 1), b)
            sv = stm_v[pl.ds(s0, GRP)]
            ov = av + sv * (bv - av) + pv * (0.5 - sv) + fcb_v[...]
            out_v[pl.ds(s0, GRP)] = ov
            return carry

        lax.fori_loop(0, GROUPS, group_body, 0)
        wait(PW - 1, 0)
        wait(PW - 1, 1)
        pltpu.sync_copy(out_v, out_hbm.at[pl.ds(base, PW)])

    return sc_kernel


_sc_kernel = _make_sc_kernel()


def kernel(wft_ics, bft_ics, stm, ft_weight, ft_bias, psqt_weight, fc_w, fc_b):
    table_bf = ft_weight.astype(jnp.bfloat16)
    psqt_f = psqt_weight.reshape(-1)
    stm_f = stm.reshape(-1)
    w1 = fc_w[0, :D]
    w2 = fc_w[0, D:]
    fcb = jnp.broadcast_to(fc_b, (16,)).astype(jnp.float32)
    out = _sc_kernel(table_bf, wft_ics, bft_ics, stm_f, psqt_f,
                     ft_bias, w1, w2, fcb)
    return out.reshape(BATCH, 1)


# speedup vs baseline: 1.1042x; 1.1013x over previous
import functools

import jax
import jax.numpy as jnp
from jax import lax
from jax.experimental import pallas as pl
from jax.experimental.pallas import tpu as pltpu
from jax.experimental.pallas import tpu_sc as plsc

N_F = 40960
D = 512
BATCH = 16384
NC = 2
NS = 16
NW = NC * NS
PW = BATCH // NW
GRP = 16
GROUPS = PW // GRP
NCH = D // 32


def _clipsq(x):
    y = jnp.minimum(jnp.maximum(x, 0.0), 1.0)
    return y * y


def _split_eo(v):
    return plsc.unpack(v, format=plsc.PackFormat.INTERLEAVED)


def _make_sc_kernel():
    mesh = plsc.VectorSubcoreMesh(core_axis_name="c", subcore_axis_name="s")

    @functools.partial(
        pl.kernel,
        mesh=mesh,
        out_type=jax.ShapeDtypeStruct((BATCH,), jnp.float32),
        compiler_params=pltpu.CompilerParams(
            needs_layout_passes=False, use_tc_tiling_on_sc=False),
        scratch_types=[
            pltpu.VMEM((PW, 64), jnp.int32),
            pltpu.VMEM((PW,), jnp.float32),
            pltpu.VMEM((PW,), jnp.float32),
            pltpu.VMEM((N_F,), jnp.float32),
            pltpu.VMEM((64, D), jnp.bfloat16),
            pltpu.VMEM((64, D), jnp.bfloat16),
            pltpu.VMEM((D,), jnp.float32),
            pltpu.VMEM((D,), jnp.float32),
            pltpu.VMEM((D,), jnp.float32),
            pltpu.VMEM((NCH, 16), jnp.float32),
            pltpu.VMEM((NCH, 16), jnp.float32),
            pltpu.VMEM((NCH, 16), jnp.float32),
            pltpu.VMEM((NCH, 16), jnp.float32),
            pltpu.VMEM((NCH, 16), jnp.float32),
            pltpu.VMEM((NCH, 16), jnp.float32),
            pltpu.VMEM((16,), jnp.float32),
            pltpu.SemaphoreType.DMA,
            pltpu.SemaphoreType.DMA,
            pltpu.SemaphoreType.DMA,
            pltpu.SemaphoreType.DMA,
        ],
    )
    def sc_kernel(table_hbm, wft_hbm, bft_hbm, stm_hbm, psqt_hbm,
                  bias_hbm, w1_hbm, w2_hbm, fcb_hbm, out_hbm,
                  idx_v, stm_v, out_v, psqt_v, rows0, rows1,
                  bias_v, w1_v, w2_v,
                  be_v, bo_v, w1e_v, w1o_v, w2e_v, w2o_v, fcb_v,
                  sem0, sem1, sem2, sem3):
        wid = lax.axis_index("s") * NC + lax.axis_index("c")
        base = wid * PW
        pltpu.sync_copy(wft_hbm.at[pl.ds(base, PW)], idx_v.at[:, pl.ds(0, 32)])
        pltpu.sync_copy(bft_hbm.at[pl.ds(base, PW)], idx_v.at[:, pl.ds(32, 32)])
        pltpu.sync_copy(stm_hbm.at[pl.ds(base, PW)], stm_v)
        pltpu.sync_copy(psqt_hbm, psqt_v)
        pltpu.sync_copy(bias_hbm, bias_v)
        pltpu.sync_copy(w1_hbm, w1_v)
        pltpu.sync_copy(w2_hbm, w2_v)
        pltpu.sync_copy(fcb_hbm, fcb_v)

        rows = (rows0, rows1)
        sems = ((sem0, sem2), (sem1, sem3))
        lanes = lax.iota(jnp.int32, 16)
        zero16 = jnp.zeros((16,), jnp.float32)

        for j in range(NCH):
            ii = lanes * 2 + (32 * j)
            be_v[j, :] = plsc.load_gather(bias_v, [ii])
            bo_v[j, :] = plsc.load_gather(bias_v, [ii + 1])
            w1e_v[j, :] = plsc.load_gather(w1_v, [ii])
            w1o_v[j, :] = plsc.load_gather(w1_v, [ii + 1])
            w2e_v[j, :] = plsc.load_gather(w2_v, [ii])
            w2o_v[j, :] = plsc.load_gather(w2_v, [ii + 1])

        def issue(i, b):
            pltpu.async_copy(table_hbm.at[idx_v.at[i, pl.ds(0, 32)]],
                             rows[b].at[pl.ds(0, 32)], sems[b][0])
            pltpu.async_copy(table_hbm.at[idx_v.at[i, pl.ds(32, 32)]],
                             rows[b].at[pl.ds(32, 32)], sems[b][1])

        def wait(i, b):
            pltpu.make_async_copy(
                table_hbm.at[idx_v.at[i, pl.ds(0, 32)]],
                rows[b].at[pl.ds(0, 32)], sems[b][0]).wait()
            pltpu.make_async_copy(
                table_hbm.at[idx_v.at[i, pl.ds(32, 32)]],
                rows[b].at[pl.ds(32, 32)], sems[b][1]).wait()

        issue(0, 0)
        issue(1, 1)

        def group_body(g, carry):
            s0 = g * GRP
            av = zero16
            bv = zero16
            pv = zero16
            for k in range(GRP):
                b = k % 2
                i = s0 + k
                wait(i, b)
                r_ref = rows[b]

                def jbody(j, acc, r_ref=r_ref):
                    cw1, cw2, cb1, cb2 = acc
                    cds = pl.ds(j * 32, 32)

                    def side(brow):
                        parts = []
                        for p in range(4):
                            a = r_ref[brow + p, cds]
                            for r in range(p + 4, 32, 4):
                                a = a + r_ref[brow + r, cds]
                            parts.append(a)
                        s = (parts[0] + parts[1]) + (parts[2] + parts[3])
                        return _split_eo(s)

                    swe, swo = side(0)
                    sbe, sbo = side(32)
                    bej = be_v[j]
                    boj = bo_v[j]
                    xwe = _clipsq(swe + bej)
                    xwo = _clipsq(swo + boj)
                    xbe = _clipsq(sbe + bej)
                    xbo = _clipsq(sbo + boj)
                    w1ej = w1e_v[j]
                    w1oj = w1o_v[j]
                    w2ej = w2e_v[j]
                    w2oj = w2o_v[j]
                    cw1 = cw1 + xwe * w1ej + xwo * w1oj
                    cw2 = cw2 + xwe * w2ej + xwo * w2oj
                    cb1 = cb1 + xbe * w1ej + xbo * w1oj
                    cb2 = cb2 + xbe * w2ej + xbo * w2oj
                    return (cw1, cw2, cb1, cb2)

                cw1, cw2, cb1, cb2 = lax.fori_loop(
                    0, NCH, jbody, (zero16, zero16, zero16, zero16))
                a_s = jnp.sum(cw1 + cb2)
                b_s = jnp.sum(cb1 + cw2)
                g0 = plsc.load_gather(psqt_v, [idx_v[i, pl.ds(0, 16)]])
                g1 = plsc.load_gather(psqt_v, [idx_v[i, pl.ds(16, 16)]])
                g2 = plsc.load_gather(psqt_v, [idx_v[i, pl.ds(32, 16)]])
                g3 = plsc.load_gather(psqt_v, [idx_v[i, pl.ds(48, 16)]])
                p_s = jnp.sum((g0 + g1) - (g2 + g3))
                m = lanes == k
                av = jnp.where(m, a_s, av)
                bv = jnp.where(m, b_s, bv)
                pv = jnp.where(m, p_s, pv)
                issue(jnp.minimum(i + 2, PW - 1), b)
            sv = stm_v[pl.ds(s0, GRP)]
            ov = av + sv * (bv - av) + pv * (0.5 - sv) + fcb_v[...]
            out_v[pl.ds(s0, GRP)] = ov
            return carry

        lax.fori_loop(0, GROUPS, group_body, 0)
        wait(PW - 1, 0)
        wait(PW - 1, 1)
        pltpu.sync_copy(out_v, out_hbm.at[pl.ds(base, PW)])

    return sc_kernel


_sc_kernel = _make_sc_kernel()


def kernel(wft_ics, bft_ics, stm, ft_weight, ft_bias, psqt_weight, fc_w, fc_b):
    table_bf = ft_weight[1:].astype(jnp.bfloat16)
    psqt_f = psqt_weight[1:].reshape(-1)
    stm_f = stm.reshape(-1)
    w1 = fc_w[0, :D]
    w2 = fc_w[0, D:]
    fcb = jnp.broadcast_to(fc_b, (16,)).astype(jnp.float32)
    out = _sc_kernel(table_bf, wft_ics, bft_ics, stm_f, psqt_f,
                     ft_bias, w1, w2, fcb)
    return out.reshape(BATCH, 1)


# speedup vs baseline: 1.3755x; 1.2457x over previous
import functools

import jax
import jax.numpy as jnp
from jax import lax
from jax.experimental import pallas as pl
from jax.experimental.pallas import tpu as pltpu
from jax.experimental.pallas import tpu_sc as plsc

N_F = 40960
D = 512
BATCH = 16384
NC = 2
NS = 16
NW = NC * NS
PW = BATCH // NW
GRP = 16
GROUPS = PW // GRP
NCH = D // 32
NBUF = 4


def _clipsq(x):
    y = jnp.minimum(jnp.maximum(x, 0.0), 1.0)
    return y * y


def _split_eo(v):
    return plsc.unpack(v, format=plsc.PackFormat.INTERLEAVED)


def _make_sc_kernel():
    mesh = plsc.VectorSubcoreMesh(core_axis_name="c", subcore_axis_name="s")

    @functools.partial(
        pl.kernel,
        mesh=mesh,
        out_type=jax.ShapeDtypeStruct((BATCH,), jnp.float32),
        compiler_params=pltpu.CompilerParams(
            needs_layout_passes=False, use_tc_tiling_on_sc=False),
        scratch_types=[
            pltpu.VMEM((PW, 32), jnp.int32),
            pltpu.VMEM((PW,), jnp.float32),
            pltpu.VMEM((PW,), jnp.float32),
            pltpu.VMEM((N_F,), jnp.float32),
            [pltpu.VMEM((64, D), jnp.bfloat16) for _ in range(NBUF)],
            [pltpu.VMEM((64,), jnp.int32) for _ in range(NBUF)],
            pltpu.VMEM((D,), jnp.float32),
            pltpu.VMEM((D,), jnp.float32),
            pltpu.VMEM((D,), jnp.float32),
            pltpu.VMEM((NCH, 16), jnp.float32),
            pltpu.VMEM((NCH, 16), jnp.float32),
            pltpu.VMEM((NCH, 16), jnp.float32),
            pltpu.VMEM((NCH, 16), jnp.float32),
            pltpu.VMEM((NCH, 16), jnp.float32),
            pltpu.VMEM((NCH, 16), jnp.float32),
            pltpu.VMEM((16,), jnp.float32),
            [pltpu.SemaphoreType.DMA for _ in range(2 * NBUF)],
        ],
    )
    def sc_kernel(table_hbm, idxp_hbm, stm_hbm, psqt_hbm,
                  bias_hbm, w1_hbm, w2_hbm, fcb_hbm, out_hbm,
                  idxp_v, stm_v, out_v, psqt_v, rows, stages,
                  bias_v, w1_v, w2_v,
                  be_v, bo_v, w1e_v, w1o_v, w2e_v, w2o_v, fcb_v, sems):
        wid = lax.axis_index("s") * NC + lax.axis_index("c")
        base = wid * PW
        pltpu.sync_copy(idxp_hbm.at[pl.ds(base, PW)], idxp_v)
        pltpu.sync_copy(stm_hbm.at[pl.ds(base, PW)], stm_v)
        pltpu.sync_copy(psqt_hbm, psqt_v)
        pltpu.sync_copy(bias_hbm, bias_v)
        pltpu.sync_copy(w1_hbm, w1_v)
        pltpu.sync_copy(w2_hbm, w2_v)
        pltpu.sync_copy(fcb_hbm, fcb_v)

        lanes = lax.iota(jnp.int32, 16)
        zero16 = jnp.zeros((16,), jnp.float32)

        for j in range(NCH):
            ii = lanes * 2 + (32 * j)
            be_v[j, :] = plsc.load_gather(bias_v, [ii])
            bo_v[j, :] = plsc.load_gather(bias_v, [ii + 1])
            w1e_v[j, :] = plsc.load_gather(w1_v, [ii])
            w1o_v[j, :] = plsc.load_gather(w1_v, [ii + 1])
            w2e_v[j, :] = plsc.load_gather(w2_v, [ii])
            w2o_v[j, :] = plsc.load_gather(w2_v, [ii + 1])

        mask16 = jnp.int32(0xFFFF)

        def build_and_issue(i, b):
            st = stages[b]
            for c in range(2):
                v = idxp_v[i, pl.ds(16 * c, 16)]
                st[pl.ds(16 * c, 16)] = v & mask16
                st[pl.ds(32 + 16 * c, 16)] = lax.shift_right_logical(v, 16)
            pltpu.async_copy(table_hbm.at[st.at[pl.ds(0, 32)]],
                             rows[b].at[pl.ds(0, 32)], sems[2 * b])
            pltpu.async_copy(table_hbm.at[st.at[pl.ds(32, 32)]],
                             rows[b].at[pl.ds(32, 32)], sems[2 * b + 1])

        def wait(b):
            st = stages[b]
            pltpu.make_async_copy(table_hbm.at[st.at[pl.ds(0, 32)]],
                                  rows[b].at[pl.ds(0, 32)], sems[2 * b]).wait()
            pltpu.make_async_copy(table_hbm.at[st.at[pl.ds(32, 32)]],
                                  rows[b].at[pl.ds(32, 32)],
                                  sems[2 * b + 1]).wait()

        for b in range(NBUF):
            build_and_issue(b, b)

        def group_body(g, carry):
            s0 = g * GRP
            av = zero16
            bv = zero16
            pv = zero16
            for k in range(GRP):
                b = k % NBUF
                i = s0 + k
                wait(b)
                r_ref = rows[b]
                st = stages[b]
                iw0 = st[pl.ds(0, 16)]
                iw1 = st[pl.ds(16, 16)]
                ib0 = st[pl.ds(32, 16)]
                ib1 = st[pl.ds(48, 16)]

                def jbody(j, acc, r_ref=r_ref):
                    cw1, cw2, cb1, cb2 = acc
                    cds = pl.ds(j * 32, 32)

                    def side(brow):
                        parts = []
                        for p in range(4):
                            a = r_ref[brow + p, cds]
                            for r in range(p + 4, 32, 4):
                                a = a + r_ref[brow + r, cds]
                            parts.append(a)
                        s = (parts[0] + parts[1]) + (parts[2] + parts[3])
                        return _split_eo(s)

                    swe, swo = side(0)
                    sbe, sbo = side(32)
                    bej = be_v[j]
                    boj = bo_v[j]
                    xwe = _clipsq(swe + bej)
                    xwo = _clipsq(swo + boj)
                    xbe = _clipsq(sbe + bej)
                    xbo = _clipsq(sbo + boj)
                    w1ej = w1e_v[j]
                    w1oj = w1o_v[j]
                    w2ej = w2e_v[j]
                    w2oj = w2o_v[j]
                    cw1 = cw1 + xwe * w1ej + xwo * w1oj
                    cw2 = cw2 + xwe * w2ej + xwo * w2oj
                    cb1 = cb1 + xbe * w1ej + xbo * w1oj
                    cb2 = cb2 + xbe * w2ej + xbo * w2oj
                    return (cw1, cw2, cb1, cb2)

                cw1, cw2, cb1, cb2 = lax.fori_loop(
                    0, NCH, jbody, (zero16, zero16, zero16, zero16))
                a_s = jnp.sum(cw1 + cb2)
                b_s = jnp.sum(cb1 + cw2)
                g0 = plsc.load_gather(psqt_v, [iw0])
                g1 = plsc.load_gather(psqt_v, [iw1])
                g2 = plsc.load_gather(psqt_v, [ib0])
                g3 = plsc.load_gather(psqt_v, [ib1])
                p_s = jnp.sum((g0 + g1) - (g2 + g3))
                m = lanes == k
                av = jnp.where(m, a_s, av)
                bv = jnp.where(m, b_s, bv)
                pv = jnp.where(m, p_s, pv)
                build_and_issue(jnp.minimum(i + NBUF, PW - 1), b)
            sv = stm_v[pl.ds(s0, GRP)]
            ov = av + sv * (bv - av) + pv * (0.5 - sv) + fcb_v[...]
            out_v[pl.ds(s0, GRP)] = ov
            return carry

        lax.fori_loop(0, GROUPS, group_body, 0)
        for b in range(NBUF):
            wait(b)
        pltpu.sync_copy(out_v, out_hbm.at[pl.ds(base, PW)])

    return sc_kernel


_sc_kernel = _make_sc_kernel()


def kernel(wft_ics, bft_ics, stm, ft_weight, ft_bias, psqt_weight, fc_w, fc_b):
    idxp = wft_ics | (bft_ics << 16)
    table_bf = ft_weight[1:].astype(jnp.bfloat16)
    psqt_f = psqt_weight[1:].reshape(-1)
    stm_f = stm.reshape(-1)
    w1 = fc_w[0, :D]
    w2 = fc_w[0, D:]
    fcb = jnp.broadcast_to(fc_b, (16,)).astype(jnp.float32)
    out = _sc_kernel(table_bf, idxp, stm_f, psqt_f,
                     ft_bias, w1, w2, fcb)
    return out.reshape(BATCH, 1)
